# Initial kernel scaffold; baseline (speedup 1.0000x reference)
import functools

import jax
import jax.numpy as jnp
from jax import lax
from jax.experimental import pallas as pl
from jax.experimental.pallas import tpu as pltpu
from jax.experimental.pallas import tpu_sc as plsc

_B = 16384
_V = 100000
_D = 4
_F_NS = 26
_F_SEQ = 4
_T = 50

_NW = 32
_BW = _B // _NW
_CB = 64
_NCHUNK = _BW // _CB

_NS_PER_CHUNK = _CB * _F_NS
_SEQ_PER_CHUNK = _CB * _F_SEQ * _T
_PAIRS_PER_CHUNK = _CB * _F_SEQ


def _body(ns_tab, seq_tab, ns_idx, seq_idx, seg, out_ns, out_seq,
          idx_ns_v, idx_seq_v, seg_v, ns_rows_v, seq_rows_v, acc_v,
          stage_v, sem_ns, sem_seq):
    wid = lax.axis_index("s") * 2 + lax.axis_index("c")

    iota = lax.iota(jnp.int32, 16)
    rowq = iota >> 2
    colr = iota & 3
    zeros16 = jnp.zeros((16,), jnp.float32)

    pltpu.sync_copy(seg, seg_v)

    for c in range(_NCHUNK):
        blk = wid * _NCHUNK + c

        pltpu.sync_copy(seq_idx.at[pl.ds(blk * 100, 100), :], idx_seq_v)
        seq_cp = pltpu.async_copy(seq_tab.at[idx_seq_v], seq_rows_v, sem_seq)
        pltpu.sync_copy(ns_idx.at[pl.ds(blk * 13, 13), :], idx_ns_v)
        ns_cp = pltpu.async_copy(ns_tab.at[idx_ns_v], ns_rows_v, sem_ns)

        def _zero(k, carry):
            plsc.store_scatter(acc_v, [rowq + 4 * k, colr], zeros16)
            return carry
        lax.fori_loop(0, _PAIRS_PER_CHUNK * _D // 16, _zero, 0)

        ns_cp.wait()
        pltpu.sync_copy(
            ns_rows_v, out_ns.at[pl.ds(blk * _NS_PER_CHUNK, _NS_PER_CHUNK), :])

        seq_cp.wait()
        pltpu.sync_copy(seq_rows_v, acc_v.at[seg_v], add=True)

        def _scale(k, carry):
            v = plsc.load_gather(acc_v, [rowq + 4 * k, colr]) * (1.0 / _T)
            stage_v[pl.ds(k * 16, 16)] = v
            return carry
        lax.fori_loop(0, _PAIRS_PER_CHUNK * _D // 16, _scale, 0)
        pltpu.sync_copy(
            stage_v,
            out_seq.at[pl.ds(blk * _PAIRS_PER_CHUNK * _D, _PAIRS_PER_CHUNK * _D)])


@jax.jit
def _sc_call(ns_tab, seq_tab, ns_idx, seq_idx, seg):
    mesh = plsc.VectorSubcoreMesh(core_axis_name="c", subcore_axis_name="s")
    f = functools.partial(
        pl.kernel,
        out_type=(
            jax.ShapeDtypeStruct((_B * _F_NS, _D), jnp.float32),
            jax.ShapeDtypeStruct((_B * _F_SEQ * _D,), jnp.float32),
        ),
        mesh=mesh,
        scratch_types=[
            pltpu.VMEM((13, 128), jnp.int32),
            pltpu.VMEM((100, 128), jnp.int32),
            pltpu.VMEM((100, 128), jnp.int32),
            pltpu.VMEM((_NS_PER_CHUNK, _D), jnp.float32),
            pltpu.VMEM((_SEQ_PER_CHUNK, _D), jnp.float32),
            pltpu.VMEM((_PAIRS_PER_CHUNK, _D), jnp.float32),
            pltpu.VMEM((_PAIRS_PER_CHUNK * _D,), jnp.float32),
            pltpu.SemaphoreType.DMA,
            pltpu.SemaphoreType.DMA,
        ],
    )(_body)
    return f(ns_tab, seq_tab, ns_idx, seq_idx, seg)


def kernel(ns_numeric, ns_sparse_idx, seq_sparse_idx, ns_tables, seq_tables):
    b = ns_sparse_idx.shape[0]
    ns_idx = (ns_sparse_idx
              + (jnp.arange(_F_NS, dtype=jnp.int32) * _V)[None, :])
    ns_idx = ns_idx.reshape(b * _F_NS // 128, 128)
    seq_idx = (seq_sparse_idx
               + (jnp.arange(_F_SEQ, dtype=jnp.int32) * _V)[None, :, None])
    seq_idx = seq_idx.reshape(b * _F_SEQ * _T // 128, 128)
    seg = (jnp.arange(_SEQ_PER_CHUNK, dtype=jnp.int32) // _T).reshape(100, 128)

    out_ns, out_seq = _sc_call(
        ns_tables.reshape(_F_NS * _V, _D),
        seq_tables.reshape(_F_SEQ * _V, _D),
        ns_idx, seq_idx, seg)

    return jnp.concatenate(
        [out_ns.reshape(b, _F_NS * _D), ns_numeric,
         out_seq.reshape(b, _F_SEQ * _D)], axis=1)


# speedup vs baseline: 10.5946x; 10.5946x over previous
import functools

import jax
import jax.numpy as jnp
from jax import lax
from jax.experimental import pallas as pl
from jax.experimental.pallas import tpu as pltpu
from jax.experimental.pallas import tpu_sc as plsc

_B = 16384
_V = 100000
_D = 4
_F_NS = 26
_F_SEQ = 4
_T = 50

_NW = 32
_BW = _B // _NW
_CB = 32
_NCHUNK = _BW // _CB

_NS_PER_CHUNK = _CB * _F_NS * _D
_SEQ_PER_CHUNK = _CB * _F_SEQ * _T * _D
_OUT_SEQ_PER_CHUNK = _CB * _F_SEQ * _D
_POOL_SLICES = _OUT_SEQ_PER_CHUNK // 16


def _body(ns_tab, seq_tab, ns_idx, seq_idx, out_ns, out_seq,
          idx_ns_v, idx_seq_v, ns_vals_v, seq_vals_v, stage_v,
          sem_ns, sem_seq):
    wid = lax.axis_index("s") * 2 + lax.axis_index("c")

    iota = lax.iota(jnp.int32, 16)
    rowq = iota >> 2
    colr = iota & 3

    for c in range(_NCHUNK):
        blk = wid * _NCHUNK + c

        pltpu.sync_copy(seq_idx.at[pl.ds(blk * _SEQ_PER_CHUNK, _SEQ_PER_CHUNK)],
                        idx_seq_v)
        seq_cp = pltpu.async_copy(seq_tab.at[idx_seq_v], seq_vals_v, sem_seq)
        pltpu.sync_copy(ns_idx.at[pl.ds(blk * _NS_PER_CHUNK, _NS_PER_CHUNK)],
                        idx_ns_v)
        ns_cp = pltpu.async_copy(ns_tab.at[idx_ns_v], ns_vals_v, sem_ns)

        ns_cp.wait()
        pltpu.sync_copy(
            ns_vals_v, out_ns.at[pl.ds(blk * _NS_PER_CHUNK, _NS_PER_CHUNK)])

        seq_cp.wait()

        def _pool(k, carry):
            base = 800 * k + 200 * rowq + colr
            def _t(t, accs):
                a0, a1 = accs
                a0 = a0 + plsc.load_gather(seq_vals_v, [base + 4 * t])
                a1 = a1 + plsc.load_gather(seq_vals_v, [base + 4 * t + 100])
                return a0, a1
            a0, a1 = lax.fori_loop(0, _T // 2, _t,
                                   (jnp.zeros((16,), jnp.float32),
                                    jnp.zeros((16,), jnp.float32)))
            stage_v[pl.ds(k * 16, 16)] = (a0 + a1) * (1.0 / _T)
            return carry
        lax.fori_loop(0, _POOL_SLICES, _pool, 0)

        pltpu.sync_copy(
            stage_v,
            out_seq.at[pl.ds(blk * _OUT_SEQ_PER_CHUNK, _OUT_SEQ_PER_CHUNK)])


@jax.jit
def _sc_call(ns_tab, seq_tab, ns_idx, seq_idx):
    mesh = plsc.VectorSubcoreMesh(
        core_axis_name="c", subcore_axis_name="s",
        num_cores=2, num_subcores=16)
    f = functools.partial(
        pl.kernel,
        out_type=(
            jax.ShapeDtypeStruct((_B * _F_NS * _D,), jnp.float32),
            jax.ShapeDtypeStruct((_B * _F_SEQ * _D,), jnp.float32),
        ),
        mesh=mesh,
        compiler_params=pltpu.CompilerParams(
            needs_layout_passes=False,
            use_tc_tiling_on_sc=False,
        ),
        scratch_types=[
            pltpu.VMEM((_NS_PER_CHUNK,), jnp.int32),
            pltpu.VMEM((_SEQ_PER_CHUNK,), jnp.int32),
            pltpu.VMEM((_NS_PER_CHUNK,), jnp.float32),
            pltpu.VMEM((_SEQ_PER_CHUNK,), jnp.float32),
            pltpu.VMEM((_OUT_SEQ_PER_CHUNK,), jnp.float32),
            pltpu.SemaphoreType.DMA,
            pltpu.SemaphoreType.DMA,
        ],
    )(_body)
    return f(ns_tab, seq_tab, ns_idx, seq_idx)


def kernel(ns_numeric, ns_sparse_idx, seq_sparse_idx, ns_tables, seq_tables):
    b = ns_sparse_idx.shape[0]
    d4 = jnp.arange(_D, dtype=jnp.int32)
    ns_base = ns_sparse_idx + (jnp.arange(_F_NS, dtype=jnp.int32) * _V)[None, :]
    ns_idx = (ns_base[:, :, None] * _D + d4).reshape(-1)
    seq_base = (seq_sparse_idx
                + (jnp.arange(_F_SEQ, dtype=jnp.int32) * _V)[None, :, None])
    seq_idx = (seq_base[:, :, :, None] * _D + d4).reshape(-1)

    out_ns, out_seq = _sc_call(
        ns_tables.reshape(-1), seq_tables.reshape(-1), ns_idx, seq_idx)

    return jnp.concatenate(
        [out_ns.reshape(b, _F_NS * _D), ns_numeric,
         out_seq.reshape(b, _F_SEQ * _D)], axis=1)


# speedup vs baseline: 12.1762x; 1.1493x over previous
import functools

import jax
import jax.numpy as jnp
from jax import lax
from jax.experimental import pallas as pl
from jax.experimental.pallas import tpu as pltpu
from jax.experimental.pallas import tpu_sc as plsc

_B = 16384
_V = 100000
_D = 4
_F_NS = 26
_F_SEQ = 4
_T = 50

_NW = 32
_BW = _B // _NW
_CB = 32
_NCHUNK = _BW // _CB

_NS_ROWS = _CB * _F_NS
_NS_ELEMS = _NS_ROWS * _D
_SEQ_ROWS = _CB * _F_SEQ * _T
_OUT_SEQ = _CB * _F_SEQ * _D


def _body(ns_tab, seq_tab, ns_idx, seq_idx, fo_seq, foe_ns, out_ns, out_seq,
          raw_ns_v, raw_seq_v, fo_seq_v, foe_ns_v, eidx_ns_v, idx4_v,
          ns_vals_v, seq_vals_v, stage_v, sem_ns, sem_seq):
    wid = lax.axis_index("s") * 2 + lax.axis_index("c")

    iota = lax.iota(jnp.int32, 16)
    rowq = iota >> 2
    colr = iota & 3

    pltpu.sync_copy(fo_seq, fo_seq_v)
    pltpu.sync_copy(foe_ns, foe_ns_v)

    for c in range(_NCHUNK):
        blk = wid * _NCHUNK + c

        pltpu.sync_copy(seq_idx.at[pl.ds(blk * _SEQ_ROWS, _SEQ_ROWS)],
                        raw_seq_v)

        def _exp_seq(s, carry):
            rv = raw_seq_v[pl.ds(s * 16, 16)]
            fo = fo_seq_v[pl.ds(s * 16, 16)]
            base = (rv << 2) + fo
            for d in range(_D):
                idx4_v[pl.ds(d * _SEQ_ROWS + s * 16, 16)] = base + d
            return carry
        lax.fori_loop(0, _SEQ_ROWS // 16, _exp_seq, 0)

        seq_cps = [
            pltpu.async_copy(
                seq_tab.at[idx4_v.at[pl.ds(d * _SEQ_ROWS, _SEQ_ROWS)]],
                seq_vals_v.at[pl.ds(d * _SEQ_ROWS, _SEQ_ROWS)],
                sem_seq)
            for d in range(_D)
        ]

        pltpu.sync_copy(ns_idx.at[pl.ds(blk * _NS_ROWS, _NS_ROWS)], raw_ns_v)

        def _exp_ns(s, carry):
            rv = plsc.load_gather(raw_ns_v, [4 * s + rowq])
            fo = foe_ns_v[pl.ds(s * 16, 16)]
            eidx_ns_v[pl.ds(s * 16, 16)] = (rv << 2) + fo
            return carry
        lax.fori_loop(0, _NS_ELEMS // 16, _exp_ns, 0)

        ns_cp = pltpu.async_copy(ns_tab.at[eidx_ns_v], ns_vals_v, sem_ns)
        ns_cp.wait()
        pltpu.sync_copy(
            ns_vals_v, out_ns.at[pl.ds(blk * _NS_ELEMS, _NS_ELEMS)])

        for cp in seq_cps:
            cp.wait()

        def _pool(k, carry):
            base = colr * _SEQ_ROWS + 200 * k + _T * rowq
            def _t(t, accs):
                a0, a1 = accs
                a0 = a0 + plsc.load_gather(seq_vals_v, [base + t])
                a1 = a1 + plsc.load_gather(seq_vals_v, [base + t + 25])
                return a0, a1
            a0, a1 = lax.fori_loop(0, _T // 2, _t,
                                   (jnp.zeros((16,), jnp.float32),
                                    jnp.zeros((16,), jnp.float32)))
            stage_v[pl.ds(k * 16, 16)] = (a0 + a1) * (1.0 / _T)
            return carry
        lax.fori_loop(0, _OUT_SEQ // 16, _pool, 0)

        pltpu.sync_copy(stage_v, out_seq.at[pl.ds(blk * _OUT_SEQ, _OUT_SEQ)])


@jax.jit
def _sc_call(ns_tab, seq_tab, ns_idx, seq_idx, fo_seq, foe_ns):
    mesh = plsc.VectorSubcoreMesh(
        core_axis_name="c", subcore_axis_name="s",
        num_cores=2, num_subcores=16)
    f = functools.partial(
        pl.kernel,
        out_type=(
            jax.ShapeDtypeStruct((_B * _F_NS * _D,), jnp.float32),
            jax.ShapeDtypeStruct((_B * _F_SEQ * _D,), jnp.float32),
        ),
        mesh=mesh,
        compiler_params=pltpu.CompilerParams(
            needs_layout_passes=False,
            use_tc_tiling_on_sc=False,
        ),
        scratch_types=[
            pltpu.VMEM((_NS_ROWS,), jnp.int32),
            pltpu.VMEM((_SEQ_ROWS,), jnp.int32),
            pltpu.VMEM((_SEQ_ROWS,), jnp.int32),
            pltpu.VMEM((_NS_ELEMS,), jnp.int32),
            pltpu.VMEM((_NS_ELEMS,), jnp.int32),
            pltpu.VMEM((_SEQ_ROWS * _D,), jnp.int32),
            pltpu.VMEM((_NS_ELEMS,), jnp.float32),
            pltpu.VMEM((_SEQ_ROWS * _D,), jnp.float32),
            pltpu.VMEM((_OUT_SEQ,), jnp.float32),
            pltpu.SemaphoreType.DMA,
            pltpu.SemaphoreType.DMA,
        ],
    )(_body)
    return f(ns_tab, seq_tab, ns_idx, seq_idx, fo_seq, foe_ns)


def kernel(ns_numeric, ns_sparse_idx, seq_sparse_idx, ns_tables, seq_tables):
    b = ns_sparse_idx.shape[0]
    j_seq = jnp.arange(_SEQ_ROWS, dtype=jnp.int32)
    fo_seq = ((j_seq // _T) % _F_SEQ) * (_V * _D)
    e_ns = jnp.arange(_NS_ELEMS, dtype=jnp.int32)
    foe_ns = ((e_ns >> 2) % _F_NS) * (_V * _D) + (e_ns & 3)

    out_ns, out_seq = _sc_call(
        ns_tables.reshape(-1), seq_tables.reshape(-1),
        ns_sparse_idx.reshape(-1), seq_sparse_idx.reshape(-1),
        fo_seq, foe_ns)

    return jnp.concatenate(
        [out_ns.reshape(b, _F_NS * _D), ns_numeric,
         out_seq.reshape(b, _F_SEQ * _D)], axis=1)


# speedup vs baseline: 54.1141x; 4.4442x over previous
import functools

import jax
import jax.numpy as jnp
from jax import lax
from jax.experimental import pallas as pl
from jax.experimental.pallas import tpu as pltpu
from jax.experimental.pallas import tpu_sc as plsc

_B = 16384
_V = 100000
_D = 4
_F_NS = 26
_F_SEQ = 4
_T = 50

_NW = 32
_BW = _B // _NW
_CB = 32
_NCHUNK = _BW // _CB

_NS_ROWS = _CB * _F_NS
_NS_ELEMS = _NS_ROWS * _D
_SEQ_ROWS = _CB * _F_SEQ * _T
_OUT_SEQ = _CB * _F_SEQ * _D


def _body(ns_t0, ns_t1, ns_t2, ns_t3, seq_t0, seq_t1, seq_t2, seq_t3,
          ns_idx, seq_idx, fo_seq, fo_ns, out_ns, out_seq,
          raw_ns_v, raw_seq_v, fo_seq_v, fo_ns_v, idx_ns_v, idx_seq_v,
          ns_vals_v, ns_out_v, seq_vals_v, stage_v, sem_ns, sem_seq):
    ns_tabs = (ns_t0, ns_t1, ns_t2, ns_t3)
    seq_tabs = (seq_t0, seq_t1, seq_t2, seq_t3)
    wid = lax.axis_index("s") * 2 + lax.axis_index("c")

    iota = lax.iota(jnp.int32, 16)

    pltpu.sync_copy(fo_seq, fo_seq_v)
    pltpu.sync_copy(fo_ns, fo_ns_v)

    for c in range(_NCHUNK):
        blk = wid * _NCHUNK + c

        pltpu.sync_copy(seq_idx.at[pl.ds(blk * _SEQ_ROWS, _SEQ_ROWS)],
                        raw_seq_v)

        def _exp_seq(s, carry):
            idx_seq_v[pl.ds(s * 16, 16)] = (raw_seq_v[pl.ds(s * 16, 16)]
                                            + fo_seq_v[pl.ds(s * 16, 16)])
            return carry
        lax.fori_loop(0, _SEQ_ROWS // 16, _exp_seq, 0)

        seq_cps = [
            pltpu.async_copy(
                seq_tabs[d].at[idx_seq_v],
                seq_vals_v.at[pl.ds(d * _SEQ_ROWS, _SEQ_ROWS)],
                sem_seq)
            for d in range(_D)
        ]

        pltpu.sync_copy(ns_idx.at[pl.ds(blk * _NS_ROWS, _NS_ROWS)], raw_ns_v)

        def _exp_ns(s, carry):
            idx_ns_v[pl.ds(s * 16, 16)] = (raw_ns_v[pl.ds(s * 16, 16)]
                                           + fo_ns_v[pl.ds(s * 16, 16)])
            return carry
        lax.fori_loop(0, _NS_ROWS // 16, _exp_ns, 0)

        ns_cps = [
            pltpu.async_copy(
                ns_tabs[d].at[idx_ns_v],
                ns_vals_v.at[pl.ds(d * _NS_ROWS, _NS_ROWS)],
                sem_ns)
            for d in range(_D)
        ]
        for cp in ns_cps:
            cp.wait()

        def _il_ns(s, carry):
            pos = (s * 16 + iota) * 4
            for d in range(_D):
                v = ns_vals_v[pl.ds(d * _NS_ROWS + s * 16, 16)]
                plsc.store_scatter(ns_out_v, [pos + d], v)
            return carry
        lax.fori_loop(0, _NS_ROWS // 16, _il_ns, 0)

        pltpu.sync_copy(
            ns_out_v, out_ns.at[pl.ds(blk * _NS_ELEMS, _NS_ELEMS)])

        for cp in seq_cps:
            cp.wait()

        rowq = iota >> 2
        colr = iota & 3
        def _pool(k, carry):
            base = colr * _SEQ_ROWS + 200 * k + _T * rowq
            def _t(t, accs):
                a0, a1 = accs
                a0 = a0 + plsc.load_gather(seq_vals_v, [base + t])
                a1 = a1 + plsc.load_gather(seq_vals_v, [base + t + 25])
                return a0, a1
            a0, a1 = lax.fori_loop(0, _T // 2, _t,
                                   (jnp.zeros((16,), jnp.float32),
                                    jnp.zeros((16,), jnp.float32)))
            stage_v[pl.ds(k * 16, 16)] = (a0 + a1) * (1.0 / _T)
            return carry
        lax.fori_loop(0, _OUT_SEQ // 16, _pool, 0)

        pltpu.sync_copy(stage_v, out_seq.at[pl.ds(blk * _OUT_SEQ, _OUT_SEQ)])


@jax.jit
def _sc_call(ns_tabs, seq_tabs, ns_idx, seq_idx, fo_seq, fo_ns):
    mesh = plsc.VectorSubcoreMesh(
        core_axis_name="c", subcore_axis_name="s",
        num_cores=2, num_subcores=16)
    f = functools.partial(
        pl.kernel,
        out_type=(
            jax.ShapeDtypeStruct((_B * _F_NS * _D,), jnp.float32),
            jax.ShapeDtypeStruct((_B * _F_SEQ * _D,), jnp.float32),
        ),
        mesh=mesh,
        compiler_params=pltpu.CompilerParams(
            needs_layout_passes=False,
            use_tc_tiling_on_sc=False,
        ),
        scratch_types=[
            pltpu.VMEM((_NS_ROWS,), jnp.int32),
            pltpu.VMEM((_SEQ_ROWS,), jnp.int32),
            pltpu.VMEM((_SEQ_ROWS,), jnp.int32),
            pltpu.VMEM((_NS_ROWS,), jnp.int32),
            pltpu.VMEM((_NS_ROWS,), jnp.int32),
            pltpu.VMEM((_SEQ_ROWS,), jnp.int32),
            pltpu.VMEM((_NS_ELEMS,), jnp.float32),
            pltpu.VMEM((_NS_ELEMS,), jnp.float32),
            pltpu.VMEM((_SEQ_ROWS * _D,), jnp.float32),
            pltpu.VMEM((_OUT_SEQ,), jnp.float32),
            pltpu.SemaphoreType.DMA,
            pltpu.SemaphoreType.DMA,
        ],
    )(_body)
    return f(*ns_tabs, *seq_tabs, ns_idx, seq_idx, fo_seq, fo_ns)


def kernel(ns_numeric, ns_sparse_idx, seq_sparse_idx, ns_tables, seq_tables):
    b = ns_sparse_idx.shape[0]
    j_seq = jnp.arange(_SEQ_ROWS, dtype=jnp.int32)
    fo_seq = ((j_seq // _T) % _F_SEQ) * _V
    j_ns = jnp.arange(_NS_ROWS, dtype=jnp.int32)
    fo_ns = (j_ns % _F_NS) * _V

    ns_tabs = tuple(ns_tables[:, :, d].reshape(-1) for d in range(_D))
    seq_tabs = tuple(seq_tables[:, :, d].reshape(-1) for d in range(_D))

    out_ns, out_seq = _sc_call(
        ns_tabs, seq_tabs,
        ns_sparse_idx.reshape(-1), seq_sparse_idx.reshape(-1),
        fo_seq, fo_ns)

    return jnp.concatenate(
        [out_ns.reshape(b, _F_NS * _D), ns_numeric,
         out_seq.reshape(b, _F_SEQ * _D)], axis=1)


# speedup vs baseline: 57.9247x; 1.0704x over previous
import functools

import jax
import jax.numpy as jnp
from jax import lax
from jax.experimental import pallas as pl
from jax.experimental.pallas import tpu as pltpu
from jax.experimental.pallas import tpu_sc as plsc

_B = 16384
_V = 100000
_D = 4
_F_NS = 26
_F_SEQ = 4
_T = 50

_NW = 32
_BW = _B // _NW
_CB = 16
_NCHUNK = _BW // _CB

_NS_ROWS = _CB * _F_NS
_NS_ELEMS = _NS_ROWS * _D
_SEQ_ROWS = _CB * _F_SEQ * _T
_OUT_SEQ = _CB * _F_SEQ * _D


def _body(ns_t0, ns_t1, ns_t2, ns_t3, seq_t0, seq_t1, seq_t2, seq_t3,
          ns_idx, seq_idx, fo_seq, fo_ns, out_ns, out_seq,
          raw_seq0, raw_seq1, idx_seq0, idx_seq1, seq_vals0, seq_vals1,
          raw_ns0, raw_ns1, idx_ns0, idx_ns1, ns_vals0, ns_vals1,
          fo_seq_v, fo_ns_v, ns_out_v, stage_v,
          sem_seq0, sem_seq1, sem_ns0, sem_ns1):
    ns_tabs = (ns_t0, ns_t1, ns_t2, ns_t3)
    seq_tabs = (seq_t0, seq_t1, seq_t2, seq_t3)
    raw_seq = (raw_seq0, raw_seq1)
    idx_seq = (idx_seq0, idx_seq1)
    seq_vals = (seq_vals0, seq_vals1)
    raw_ns = (raw_ns0, raw_ns1)
    idx_ns = (idx_ns0, idx_ns1)
    ns_vals = (ns_vals0, ns_vals1)
    sem_seq = (sem_seq0, sem_seq1)
    sem_ns = (sem_ns0, sem_ns1)

    wid = lax.axis_index("s") * 2 + lax.axis_index("c")

    iota = lax.iota(jnp.int32, 16)
    rowq = iota >> 2
    colr = iota & 3

    pltpu.sync_copy(fo_seq, fo_seq_v)
    pltpu.sync_copy(fo_ns, fo_ns_v)

    def stage_and_fire(c, slot):
        blk = wid * _NCHUNK + c
        pltpu.sync_copy(seq_idx.at[pl.ds(blk * _SEQ_ROWS, _SEQ_ROWS)],
                        raw_seq[slot])
        def _exp_seq(s, carry):
            idx_seq[slot][pl.ds(s * 16, 16)] = (
                raw_seq[slot][pl.ds(s * 16, 16)]
                + fo_seq_v[pl.ds(s * 16, 16)])
            return carry
        lax.fori_loop(0, _SEQ_ROWS // 16, _exp_seq, 0)
        seq_cps = [
            pltpu.async_copy(
                seq_tabs[d].at[idx_seq[slot]],
                seq_vals[slot].at[pl.ds(d * _SEQ_ROWS, _SEQ_ROWS)],
                sem_seq[slot])
            for d in range(_D)
        ]

        pltpu.sync_copy(ns_idx.at[pl.ds(blk * _NS_ROWS, _NS_ROWS)],
                        raw_ns[slot])
        def _exp_ns(s, carry):
            idx_ns[slot][pl.ds(s * 16, 16)] = (
                raw_ns[slot][pl.ds(s * 16, 16)]
                + fo_ns_v[pl.ds(s * 16, 16)])
            return carry
        lax.fori_loop(0, _NS_ROWS // 16, _exp_ns, 0)
        ns_cps = [
            pltpu.async_copy(
                ns_tabs[d].at[idx_ns[slot]],
                ns_vals[slot].at[pl.ds(d * _NS_ROWS, _NS_ROWS)],
                sem_ns[slot])
            for d in range(_D)
        ]
        return seq_cps, ns_cps

    def drain(c, slot, cps):
        blk = wid * _NCHUNK + c
        seq_cps, ns_cps = cps
        for cp in ns_cps:
            cp.wait()
        def _il_ns(s, carry):
            pos = (s * 16 + iota) * 4
            for d in range(_D):
                v = ns_vals[slot][pl.ds(d * _NS_ROWS + s * 16, 16)]
                plsc.store_scatter(ns_out_v, [pos + d], v)
            return carry
        lax.fori_loop(0, _NS_ROWS // 16, _il_ns, 0)
        pltpu.sync_copy(ns_out_v,
                        out_ns.at[pl.ds(blk * _NS_ELEMS, _NS_ELEMS)])

        for cp in seq_cps:
            cp.wait()
        def _pool(k, carry):
            base = colr * _SEQ_ROWS + 200 * k + _T * rowq
            def _t(t, accs):
                a0, a1 = accs
                a0 = a0 + plsc.load_gather(seq_vals[slot], [base + t])
                a1 = a1 + plsc.load_gather(seq_vals[slot], [base + t + 25])
                return a0, a1
            a0, a1 = lax.fori_loop(0, _T // 2, _t,
                                   (jnp.zeros((16,), jnp.float32),
                                    jnp.zeros((16,), jnp.float32)))
            stage_v[pl.ds(k * 16, 16)] = (a0 + a1) * (1.0 / _T)
            return carry
        lax.fori_loop(0, _OUT_SEQ // 16, _pool, 0)
        pltpu.sync_copy(stage_v,
                        out_seq.at[pl.ds(blk * _OUT_SEQ, _OUT_SEQ)])

    cps = stage_and_fire(0, 0)
    for c in range(_NCHUNK):
        slot = c & 1
        if c + 1 < _NCHUNK:
            next_cps = stage_and_fire(c + 1, 1 - slot)
        drain(c, slot, cps)
        if c + 1 < _NCHUNK:
            cps = next_cps


@jax.jit
def _sc_call(ns_tabs, seq_tabs, ns_idx, seq_idx, fo_seq, fo_ns):
    mesh = plsc.VectorSubcoreMesh(
        core_axis_name="c", subcore_axis_name="s",
        num_cores=2, num_subcores=16)
    f = functools.partial(
        pl.kernel,
        out_type=(
            jax.ShapeDtypeStruct((_B * _F_NS * _D,), jnp.float32),
            jax.ShapeDtypeStruct((_B * _F_SEQ * _D,), jnp.float32),
        ),
        mesh=mesh,
        compiler_params=pltpu.CompilerParams(
            needs_layout_passes=False,
            use_tc_tiling_on_sc=False,
        ),
        scratch_types=(
            [pltpu.VMEM((_SEQ_ROWS,), jnp.int32)] * 4
            + [pltpu.VMEM((_SEQ_ROWS * _D,), jnp.float32)] * 2
            + [pltpu.VMEM((_NS_ROWS,), jnp.int32)] * 4
            + [pltpu.VMEM((_NS_ELEMS,), jnp.float32)] * 2
            + [pltpu.VMEM((_SEQ_ROWS,), jnp.int32),
               pltpu.VMEM((_NS_ROWS,), jnp.int32),
               pltpu.VMEM((_NS_ELEMS,), jnp.float32),
               pltpu.VMEM((_OUT_SEQ,), jnp.float32)]
            + [pltpu.SemaphoreType.DMA] * 4
        ),
    )(_body)
    return f(*ns_tabs, *seq_tabs, ns_idx, seq_idx, fo_seq, fo_ns)


def kernel(ns_numeric, ns_sparse_idx, seq_sparse_idx, ns_tables, seq_tables):
    b = ns_sparse_idx.shape[0]
    j_seq = jnp.arange(_SEQ_ROWS, dtype=jnp.int32)
    fo_seq = ((j_seq // _T) % _F_SEQ) * _V
    j_ns = jnp.arange(_NS_ROWS, dtype=jnp.int32)
    fo_ns = (j_ns % _F_NS) * _V

    ns_tabs = tuple(ns_tables[:, :, d].reshape(-1) for d in range(_D))
    seq_tabs = tuple(seq_tables[:, :, d].reshape(-1) for d in range(_D))

    out_ns, out_seq = _sc_call(
        ns_tabs, seq_tabs,
        ns_sparse_idx.reshape(-1), seq_sparse_idx.reshape(-1),
        fo_seq, fo_ns)

    return jnp.concatenate(
        [out_ns.reshape(b, _F_NS * _D), ns_numeric,
         out_seq.reshape(b, _F_SEQ * _D)], axis=1)
